# baseline (speedup 1.0000x reference)
import functools

import jax
import jax.numpy as jnp
from jax import lax
from jax.experimental import pallas as pl
from jax.experimental.pallas import tpu as pltpu, tpu_sc as plsc

_NC = 2
_NS = 16
_NW = _NC * _NS
_L = 16
_CH = 128


def _sc_body(x_hbm, wb_hbm, out_hbm, xbuf, obuf, wbv):
    rows = x_hbm.shape[0]
    rows_per_w = rows // _NW
    wid = lax.axis_index("s") * _NC + lax.axis_index("c")
    base = wid * rows_per_w

    pltpu.sync_copy(wb_hbm, wbv)

    iota = lax.iota(jnp.int32, _L)

    def do_chunk(c, _):
        pltpu.sync_copy(x_hbm.at[pl.ds(base + c * _CH, _CH)], xbuf)
        for g in range(_CH // _L):
            row_ids = iota + (g * _L)

            def fstep(off):
                def step(f, accs):
                    col = jnp.full((_L,), 0, jnp.int32) + (f + off)
                    xv = plsc.load_gather(xbuf, [row_ids, col])
                    return tuple(
                        accs[d] + xv * wbv[(f + off) * _L + d]
                        for d in range(_L)
                    )
                return step

            zero = jnp.zeros((_L,), jnp.float32)
            acc_r = lax.fori_loop(0, 128, fstep(0), (zero,) * _L,
                                  unroll=4)
            acc_i = lax.fori_loop(0, 128, fstep(128), (zero,) * _L,
                                  unroll=4)
            for d in range(_L):
                plsc.store_scatter(
                    obuf, [row_ids, jnp.full((_L,), d, jnp.int32)], acc_r[d])
                plsc.store_scatter(
                    obuf, [row_ids, jnp.full((_L,), d + _L, jnp.int32)],
                    acc_i[d])
        pltpu.sync_copy(obuf, out_hbm.at[pl.ds(base + c * _CH, _CH)])
        return ()

    lax.fori_loop(0, rows_per_w // _CH, do_chunk, ())


def kernel(inputs, lookup_table_real, lookup_table_imag):
    B, T, F2 = inputs.shape
    half = lookup_table_real.shape[1]
    D = 2 * half
    rows = B * T
    x = inputs.reshape(rows, F2)

    w = jnp.concatenate([lookup_table_real, lookup_table_imag], axis=0)
    wb = jnp.broadcast_to(w.reshape(F2 * half, 1), (F2 * half, _L))
    wb = jnp.asarray(wb)

    mesh = plsc.VectorSubcoreMesh(core_axis_name="c", subcore_axis_name="s")

    fe = pl.kernel(
        _sc_body,
        out_type=jax.ShapeDtypeStruct((rows, D), jnp.float32),
        mesh=mesh,
        scratch_types=[
            pltpu.VMEM((_CH, F2), jnp.float32),
            pltpu.VMEM((_CH, D), jnp.float32),
            pltpu.VMEM((F2 * half, _L), jnp.float32),
        ],
        compiler_params=pltpu.CompilerParams(use_tc_tiling_on_sc=False, needs_layout_passes=False),
    )
    out = fe(x, wb)
    return out.reshape(B, T, D)
